# baseline (speedup 1.0000x reference)
import functools

import jax
import jax.numpy as jnp
import numpy as np
from jax.experimental import pallas as pl
from jax.experimental.pallas import tpu as pltpu

NUM_EMB = 8192
EMB_DIM = 256
NUM_SAMPLES = 5
COMMIT = 0.25
N_TOK = 4608

TN = 256
KC = 1024
N_BLOCKS = N_TOK // TN
N_CHUNKS = NUM_EMB // KC


def _i32(v):
    return np.int32(np.uint32(v).astype(np.int64) - (1 << 32)
                    if np.uint32(v) >= (1 << 31) else np.uint32(v))


_K0 = np.int32(0)
_K1 = np.int32(42)
_K2 = _i32(0 ^ 42 ^ 0x1BD11BDA)
_ROT_A = (13, 15, 26, 6)
_ROT_B = (17, 29, 16, 24)
_TINY = np.float32(np.finfo(np.float32).tiny)


def _rotl(x, d):
    return jax.lax.bitwise_or(
        jax.lax.shift_left(x, np.int32(d)),
        jax.lax.shift_right_logical(x, np.int32(32 - d)))


def _four_rounds(v0, v1, rots):
    for r in rots:
        v0 = v0 + v1
        v1 = _rotl(v1, r)
        v1 = jax.lax.bitwise_xor(v0, v1)
    return v0, v1


def _threefry_bits(ctr):
    v0 = jnp.full(ctr.shape, _K0, jnp.int32)
    v1 = ctr + _K1
    v0, v1 = _four_rounds(v0, v1, _ROT_A)
    v0 = v0 + _K1
    v1 = v1 + _i32(int(_K2) + 1)
    v0, v1 = _four_rounds(v0, v1, _ROT_B)
    v0 = v0 + _K2
    v1 = v1 + _i32(int(_K0) + 2)
    v0, v1 = _four_rounds(v0, v1, _ROT_A)
    v0 = v0 + _K0
    v1 = v1 + _i32(int(_K1) + 3)
    v0, v1 = _four_rounds(v0, v1, _ROT_B)
    v0 = v0 + _K1
    v1 = v1 + _i32(int(_K2) + 4)
    v0, v1 = _four_rounds(v0, v1, _ROT_A)
    v0 = v0 + _K2
    v1 = v1 + _i32(int(_K0) + 5)
    return jax.lax.bitwise_xor(v0, v1)


def _gumbel_from_bits(bits):
    fb = jax.lax.bitwise_or(
        jax.lax.shift_right_logical(bits, np.int32(9)),
        np.int32(0x3F800000))
    u = jax.lax.bitcast_convert_type(fb, jnp.float32) - np.float32(1.0)
    u = jnp.maximum(u, _TINY)
    return -jnp.log(-jnp.log(u))


def _sampler_body(x_ref, w_ref, out_ref, logits_ref):
    b = pl.program_id(0)
    s = pl.program_id(1)

    @pl.when(s == 0)
    def _():
        wv = w_ref[...]
        xb = x_ref[...]
        w2 = jnp.sum(wv * wv, axis=1)
        x2 = jnp.sum(xb * xb, axis=1)
        mm = jax.lax.dot_general(
            wv, xb, (((1,), (1,)), ((), ())),
            preferred_element_type=jnp.float32)
        dist = (w2[:, None] + x2[None, :]) - np.float32(2.0) * mm
        logits_ref[...] = -dist

    tj = jax.lax.broadcasted_iota(jnp.int32, (KC, TN), 0)
    ti = jax.lax.broadcasted_iota(jnp.int32, (KC, TN), 1)
    row_ctr = ti * np.int32(NUM_EMB) + tj
    s_base = s * np.int32(N_TOK * NUM_EMB) + b * np.int32(TN * NUM_EMB)

    def chunk_step(c, carry):
        best, bidx = carry
        ctr = row_ctr + (s_base + c * np.int32(KC))
        g = _gumbel_from_bits(_threefry_bits(ctr))
        lg = logits_ref[pl.ds(c * KC, KC), :]
        val = g + lg
        cm = jnp.max(val, axis=0, keepdims=True)
        jglob = tj + c * np.int32(KC)
        cand = jnp.where(val == cm, jglob, np.int32(1 << 30))
        cidx = jnp.min(cand, axis=0, keepdims=True)
        take = cm > best
        return (jnp.where(take, cm, best), jnp.where(take, cidx, bidx))

    init = (jnp.full((1, TN), -np.inf, jnp.float32),
            jnp.zeros((1, TN), jnp.int32))
    _, bidx = jax.lax.fori_loop(0, N_CHUNKS, chunk_step, init)
    out_ref[...] = bidx[None]


def _sample_tokens(xf, w):
    out = pl.pallas_call(
        _sampler_body,
        grid=(N_BLOCKS, NUM_SAMPLES),
        in_specs=[
            pl.BlockSpec((TN, EMB_DIM), lambda b, s: (b, 0)),
            pl.BlockSpec((NUM_EMB, EMB_DIM), lambda b, s: (0, 0)),
        ],
        out_specs=pl.BlockSpec((1, 1, TN), lambda b, s: (s, 0, b)),
        out_shape=jax.ShapeDtypeStruct((NUM_SAMPLES, 1, N_TOK), jnp.int32),
        scratch_shapes=[pltpu.VMEM((NUM_EMB, TN), jnp.float32)],
        compiler_params=pltpu.CompilerParams(
            dimension_semantics=("arbitrary", "arbitrary")),
    )(xf, w)
    return out.reshape(NUM_SAMPLES, N_TOK)


def kernel(x, W):
    xf = x.reshape(-1, EMB_DIM)
    samples = _sample_tokens(xf, W)

    rows = jnp.broadcast_to(jnp.arange(N_TOK), (NUM_SAMPLES, N_TOK)).reshape(-1)
    encodings = jnp.zeros((N_TOK, NUM_EMB), dtype=jnp.float32).at[
        rows, samples.reshape(-1)].add(1.0 / NUM_SAMPLES)
    quantized = jnp.mean(jnp.take(W, samples, axis=0), axis=0)
    quantized = quantized.reshape(x.shape)
    e_latent_loss = jnp.mean((x - jax.lax.stop_gradient(quantized)) ** 2)
    loss = COMMIT * e_latent_loss
    quantized_st = x + jax.lax.stop_gradient(quantized - x)
    avg_probs = jnp.mean(encodings, axis=0)
    perplexity = jnp.exp(-jnp.sum(avg_probs * jnp.log(avg_probs + 1e-10)))
    return (quantized_st, loss, perplexity)
